# baseline (speedup 1.0000x reference)
import jax
import jax.numpy as jnp
import numpy as np
from jax.experimental import pallas as pl
from jax.experimental.pallas import tpu as pltpu

FEAT_STRIDE = 16
POST_NMS_TOP_N = 300
_H = 64
_W = 64
_ROWS = 288
_LANES = 128
_G = 36
_GPAD = 40


def _mk(ws, hs, x_ctr, y_ctr):
    ws = np.asarray(ws, dtype=np.float64).reshape(-1, 1)
    hs = np.asarray(hs, dtype=np.float64).reshape(-1, 1)
    return np.hstack((x_ctr - 0.5 * (ws - 1), y_ctr - 0.5 * (hs - 1),
                      x_ctr + 0.5 * (ws - 1), y_ctr + 0.5 * (hs - 1)))


def _gen_anchors(base_size=16, ratios=(0.5, 1.0, 2.0), scales=(8, 16, 32),
                 time_dim=(16,)):
    ratios = np.array(ratios)
    scales = np.array(scales)
    x_ctr = 0.5 * (base_size - 1)
    y_ctr = 0.5 * (base_size - 1)
    size = float(base_size * base_size)
    size_ratios = size / ratios
    ws = np.round(np.sqrt(size_ratios))
    hs = np.round(ws * ratios)
    ratio_anchors = _mk(ws, hs, x_ctr, y_ctr)
    all_a = []
    for a in ratio_anchors:
        w = a[2] - a[0] + 1.0
        h = a[3] - a[1] + 1.0
        xc = a[0] + 0.5 * (w - 1)
        yc = a[1] + 0.5 * (h - 1)
        all_a.append(_mk(w * scales, h * scales, xc, yc))
    a2d = np.vstack(all_a)
    out = []
    for t in time_dim:
        n = a2d.shape[0]
        out.append(np.hstack((a2d[:, 0:2], np.zeros((n, 1)), a2d[:, 2:4],
                              np.full((n, 1), float(t) - 1.0))))
    return np.vstack(out).astype(np.float32)


_ANC = _gen_anchors()
_A = _ANC.shape[0]
_Z1 = float(_ANC[0, 2])
_Z2 = float(_ANC[0, 5])


def _anchor_table():
    w9 = _ANC[:, 3] - _ANC[:, 0] + 1.0
    h9 = _ANC[:, 4] - _ANC[:, 1] + 1.0
    cx9 = _ANC[:, 0] + 0.5 * w9
    cy9 = _ANC[:, 1] + 0.5 * h9
    return np.stack([w9, h9, cx9, cy9]).astype(np.float32)


_ANC_TAB = _anchor_table()


_BPI = 2


def _proposal_kernel(im_ref, anc_ref, sc_ref, dx_ref, dy_ref, dw_ref, dh_ref,
                     out_ref, buf):
    pid = pl.program_id(0)

    for bl in range(_BPI):
        buf[bl] = sc_ref[bl]

    idx2d = jax.lax.broadcasted_iota(jnp.int32, (_ROWS, _LANES), 0) * _LANES \
        + jax.lax.broadcasted_iota(jnp.int32, (_ROWS, _LANES), 1)
    lane = jax.lax.broadcasted_iota(jnp.int32, (1, _LANES), 1)
    big = jnp.int32(2**30)
    xmaxs = [im_ref[pid * _BPI + bl, 1] - 1.0 for bl in range(_BPI)]
    ymaxs = [im_ref[pid * _BPI + bl, 0] - 1.0 for bl in range(_BPI)]
    bcols = [(pid * _BPI + bl).astype(jnp.float32) for bl in range(_BPI)]

    def body(j, _):
        ms, rs, cs = [], [], []
        for bl in range(_BPI):
            data = buf[bl]
            m = jnp.max(data)
            cand = jnp.min(jnp.where(data == m, idx2d, big))
            r = cand // _LANES
            ms.append(m)
            rs.append(r)
            cs.append(cand - r * _LANES)
        for bl in range(_BPI):
            sel = lane == cs[bl]
            buf[bl, pl.ds(rs[bl], 1), :] = jnp.where(
                sel, -jnp.inf, buf[bl, pl.ds(rs[bl], 1), :])

        for bl in range(_BPI):
            xmax = xmaxs[bl]
            ymax = ymaxs[bl]
            m, r, c = ms[bl], rs[bl], cs[bl]
            sel = lane == c

            def gat(ref, bl=bl, r=r, sel=sel):
                return jnp.sum(jnp.where(sel, ref[bl, pl.ds(r, 1), :], 0.0))

            dx, dy, dw, dh = gat(dx_ref), gat(dy_ref), gat(dw_ref), gat(dh_ref)

            flat = r * _LANES + c
            k = flat // _A
            a = flat - k * _A
            wsh = (k % _W).astype(jnp.float32) * float(FEAT_STRIDE)
            hsh = (k // _W).astype(jnp.float32) * float(FEAT_STRIDE)
            aw = anc_ref[0, a]
            ah = anc_ref[1, a]
            acx = anc_ref[2, a] + wsh
            acy = anc_ref[3, a] + hsh

            pcx = dx * aw + acx
            pcy = dy * ah + acy
            pw = jnp.exp(dw) * aw
            ph = jnp.exp(dh) * ah
            x1 = jnp.clip(pcx - 0.5 * pw, 0.0, xmax)
            y1 = jnp.clip(pcy - 0.5 * ph, 0.0, ymax)
            x2 = jnp.clip(pcx + 0.5 * pw, 0.0, xmax)
            y2 = jnp.clip(pcy + 0.5 * ph, 0.0, ymax)

            row = jnp.full((1, _LANES), 0.0, dtype=jnp.float32)
            for li, v in enumerate((bcols[bl], x1, y1,
                                    jnp.float32(_Z1), x2, y2,
                                    jnp.float32(_Z2), m)):
                row = jnp.where(lane == li, v, row)
            out_ref[bl, pl.ds(j, 1), :] = row
        return 0

    jax.lax.fori_loop(0, POST_NMS_TOP_N, body, 0)


@jax.jit
def _run(scores, bbox_frame, im_info):
    B = scores.shape[0]
    sc = jnp.transpose(scores[:, _A:, :, :], (0, 2, 3, 1)).reshape(
        B, _ROWS, _LANES)
    bb = jnp.transpose(bbox_frame, (0, 2, 3, 1)).reshape(B, _ROWS * _LANES, 4)
    planes = [bb[..., d].reshape(B, _ROWS, _LANES) for d in range(4)]

    data_spec = pl.BlockSpec((_BPI, _ROWS, _LANES), lambda g: (g, 0, 0))
    out = pl.pallas_call(
        _proposal_kernel,
        grid=(B // _BPI,),
        in_specs=[pl.BlockSpec(memory_space=pltpu.SMEM)] * 2
        + [data_spec] * 5,
        out_specs=pl.BlockSpec((_BPI, 304, _LANES), lambda g: (g, 0, 0)),
        out_shape=jax.ShapeDtypeStruct((B, 304, _LANES), jnp.float32),
        scratch_shapes=[pltpu.VMEM((_BPI, _ROWS, _LANES), jnp.float32)],
        compiler_params=pltpu.CompilerParams(
            dimension_semantics=("parallel",)),
    )(im_info, jnp.asarray(_ANC_TAB), sc, *planes)
    return out[:, :POST_NMS_TOP_N, :8]


def kernel(scores, bbox_frame, im_info, time_dim):
    return _run(scores, bbox_frame, im_info)


# speedup vs baseline: 1.0622x; 1.0622x over previous
import jax
import jax.numpy as jnp
import numpy as np
from jax.experimental import pallas as pl
from jax.experimental.pallas import tpu as pltpu

FEAT_STRIDE = 16
POST_NMS_TOP_N = 300
_H = 64
_W = 64
_ROWS = 288
_LANES = 128
_G = 36
_GPAD = 40


def _mk(ws, hs, x_ctr, y_ctr):
    ws = np.asarray(ws, dtype=np.float64).reshape(-1, 1)
    hs = np.asarray(hs, dtype=np.float64).reshape(-1, 1)
    return np.hstack((x_ctr - 0.5 * (ws - 1), y_ctr - 0.5 * (hs - 1),
                      x_ctr + 0.5 * (ws - 1), y_ctr + 0.5 * (hs - 1)))


def _gen_anchors(base_size=16, ratios=(0.5, 1.0, 2.0), scales=(8, 16, 32),
                 time_dim=(16,)):
    ratios = np.array(ratios)
    scales = np.array(scales)
    x_ctr = 0.5 * (base_size - 1)
    y_ctr = 0.5 * (base_size - 1)
    size = float(base_size * base_size)
    size_ratios = size / ratios
    ws = np.round(np.sqrt(size_ratios))
    hs = np.round(ws * ratios)
    ratio_anchors = _mk(ws, hs, x_ctr, y_ctr)
    all_a = []
    for a in ratio_anchors:
        w = a[2] - a[0] + 1.0
        h = a[3] - a[1] + 1.0
        xc = a[0] + 0.5 * (w - 1)
        yc = a[1] + 0.5 * (h - 1)
        all_a.append(_mk(w * scales, h * scales, xc, yc))
    a2d = np.vstack(all_a)
    out = []
    for t in time_dim:
        n = a2d.shape[0]
        out.append(np.hstack((a2d[:, 0:2], np.zeros((n, 1)), a2d[:, 2:4],
                              np.full((n, 1), float(t) - 1.0))))
    return np.vstack(out).astype(np.float32)


_ANC = _gen_anchors()
_A = _ANC.shape[0]
_Z1 = float(_ANC[0, 2])
_Z2 = float(_ANC[0, 5])


def _anchor_table():
    w9 = _ANC[:, 3] - _ANC[:, 0] + 1.0
    h9 = _ANC[:, 4] - _ANC[:, 1] + 1.0
    cx9 = _ANC[:, 0] + 0.5 * w9
    cy9 = _ANC[:, 1] + 0.5 * h9
    return np.stack([w9, h9, cx9, cy9]).astype(np.float32)


_ANC_TAB = _anchor_table()


_BPI = 2


def _proposal_kernel(im_ref, anc_ref, sc_ref, dx_ref, dy_ref, dw_ref, dh_ref,
                     out_ref, buf):
    pid = pl.program_id(0)

    for bl in range(_BPI):
        buf[bl] = sc_ref[bl]

    idx2d = jax.lax.broadcasted_iota(jnp.int32, (_ROWS, _LANES), 0) * _LANES \
        + jax.lax.broadcasted_iota(jnp.int32, (_ROWS, _LANES), 1)
    lane = jax.lax.broadcasted_iota(jnp.int32, (1, _LANES), 1)
    big = jnp.int32(2**30)
    xmaxs = [im_ref[pid * _BPI + bl, 1] - 1.0 for bl in range(_BPI)]
    ymaxs = [im_ref[pid * _BPI + bl, 0] - 1.0 for bl in range(_BPI)]
    bcols = [(pid * _BPI + bl).astype(jnp.float32) for bl in range(_BPI)]

    def body(j2, _):
        for jj in range(2):
            j = j2 * 2 + jj
            _extract_one(j)
        return 0

    def _extract_one(j):
        ms, rs, cs = [], [], []
        for bl in range(_BPI):
            data = buf[bl]
            m = jnp.max(data)
            cand = jnp.min(jnp.where(data == m, idx2d, big))
            r = cand // _LANES
            ms.append(m)
            rs.append(r)
            cs.append(cand - r * _LANES)
        for bl in range(_BPI):
            sel = lane == cs[bl]
            buf[bl, pl.ds(rs[bl], 1), :] = jnp.where(
                sel, -jnp.inf, buf[bl, pl.ds(rs[bl], 1), :])

        for bl in range(_BPI):
            xmax = xmaxs[bl]
            ymax = ymaxs[bl]
            m, r, c = ms[bl], rs[bl], cs[bl]
            sel = lane == c

            def gat(ref, bl=bl, r=r, sel=sel):
                return jnp.sum(jnp.where(sel, ref[bl, pl.ds(r, 1), :], 0.0))

            dx, dy, dw, dh = gat(dx_ref), gat(dy_ref), gat(dw_ref), gat(dh_ref)

            flat = r * _LANES + c
            k = flat // _A
            a = flat - k * _A
            wsh = (k % _W).astype(jnp.float32) * float(FEAT_STRIDE)
            hsh = (k // _W).astype(jnp.float32) * float(FEAT_STRIDE)
            aw = anc_ref[0, a]
            ah = anc_ref[1, a]
            acx = anc_ref[2, a] + wsh
            acy = anc_ref[3, a] + hsh

            pcx = dx * aw + acx
            pcy = dy * ah + acy
            pw = jnp.exp(dw) * aw
            ph = jnp.exp(dh) * ah
            x1 = jnp.clip(pcx - 0.5 * pw, 0.0, xmax)
            y1 = jnp.clip(pcy - 0.5 * ph, 0.0, ymax)
            x2 = jnp.clip(pcx + 0.5 * pw, 0.0, xmax)
            y2 = jnp.clip(pcy + 0.5 * ph, 0.0, ymax)

            row = jnp.full((1, _LANES), 0.0, dtype=jnp.float32)
            for li, v in enumerate((bcols[bl], x1, y1,
                                    jnp.float32(_Z1), x2, y2,
                                    jnp.float32(_Z2), m)):
                row = jnp.where(lane == li, v, row)
            out_ref[bl, pl.ds(j, 1), :] = row

    jax.lax.fori_loop(0, POST_NMS_TOP_N // 2, body, 0)


@jax.jit
def _run(scores, bbox_frame, im_info):
    B = scores.shape[0]
    sc = jnp.transpose(scores[:, _A:, :, :], (0, 2, 3, 1)).reshape(
        B, _ROWS, _LANES)
    bb = jnp.transpose(bbox_frame, (0, 2, 3, 1)).reshape(B, _ROWS * _LANES, 4)
    planes = [bb[..., d].reshape(B, _ROWS, _LANES) for d in range(4)]

    data_spec = pl.BlockSpec((_BPI, _ROWS, _LANES), lambda g: (g, 0, 0))
    out = pl.pallas_call(
        _proposal_kernel,
        grid=(B // _BPI,),
        in_specs=[pl.BlockSpec(memory_space=pltpu.SMEM)] * 2
        + [data_spec] * 5,
        out_specs=pl.BlockSpec((_BPI, 304, _LANES), lambda g: (g, 0, 0)),
        out_shape=jax.ShapeDtypeStruct((B, 304, _LANES), jnp.float32),
        scratch_shapes=[pltpu.VMEM((_BPI, _ROWS, _LANES), jnp.float32)],
        compiler_params=pltpu.CompilerParams(
            dimension_semantics=("parallel",)),
    )(im_info, jnp.asarray(_ANC_TAB), sc, *planes)
    return out[:, :POST_NMS_TOP_N, :8]


def kernel(scores, bbox_frame, im_info, time_dim):
    return _run(scores, bbox_frame, im_info)
